# speedup vs baseline: 1.0223x; 1.0223x over previous
import jax
import jax.numpy as jnp
from jax.experimental import pallas as pl
from jax.experimental.pallas import tpu as pltpu


def _mlp_body(idx_ref, emb_ref, w1_ref, b1_ref, w2_ref, b2_ref, out_ref, h_ref):
    T, V = idx_ref.shape[0], emb_ref.shape[0]

    @pl.when(pl.program_id(0) == 0)
    def _compute_h():
        iota = jax.lax.broadcasted_iota(jnp.int32, (T, V), 1)
        onehot = jnp.where(iota == idx_ref[...], 1.0, 0.0).astype(jnp.float32)
        x = jnp.dot(onehot, emb_ref[...], preferred_element_type=jnp.float32)
        h = jnp.dot(x, w1_ref[...], preferred_element_type=jnp.float32)
        h_ref[...] = jnp.tanh(h + b1_ref[...])

    out_ref[...] = (
        jnp.dot(h_ref[...], w2_ref[...], preferred_element_type=jnp.float32)
        + b2_ref[...]
    )


def kernel(prefix, emb, W1, b1, W2, b2):
    B, L = prefix.shape
    V, D = emb.shape
    H = W1.shape[1]
    N = W2.shape[1]
    T = B * L

    TN = 2048
    grid = (N // TN,)

    idx = prefix.reshape(T, 1).astype(jnp.int32)

    out = pl.pallas_call(
        _mlp_body,
        grid=grid,
        in_specs=[
            pl.BlockSpec((T, 1), lambda j: (0, 0)),
            pl.BlockSpec((V, D), lambda j: (0, 0)),
            pl.BlockSpec((D, H), lambda j: (0, 0)),
            pl.BlockSpec((1, H), lambda j: (0, 0)),
            pl.BlockSpec((H, TN), lambda j: (0, j)),
            pl.BlockSpec((1, TN), lambda j: (0, j)),
        ],
        out_specs=pl.BlockSpec((T, TN), lambda j: (0, j)),
        out_shape=jax.ShapeDtypeStruct((T, N), jnp.float32),
        scratch_shapes=[pltpu.VMEM((T, H), jnp.float32)],
    )(idx, emb, W1, b1.reshape(1, H), W2, b2.reshape(1, N))

    return out.reshape(B, L, N)


# speedup vs baseline: 1.0304x; 1.0079x over previous
import jax
import jax.numpy as jnp
from jax.experimental import pallas as pl
from jax.experimental.pallas import tpu as pltpu


def _mlp_body(idx_ref, emb_ref, w1_ref, b1_ref, w2_ref, b2_ref, out_ref,
              htab_ref, oh_ref):
    T, V = idx_ref.shape[0], emb_ref.shape[0]

    @pl.when(pl.program_id(0) == 0)
    def _compute_h():
        h = jnp.dot(emb_ref[...], w1_ref[...], preferred_element_type=jnp.float32)
        htab_ref[...] = jnp.tanh(h + b1_ref[...])
        iota = jax.lax.broadcasted_iota(jnp.int32, (T, V), 1)
        oh_ref[...] = jnp.where(iota == idx_ref[...], 1.0, 0.0).astype(jnp.float32)

    m = jnp.dot(htab_ref[...], w2_ref[...], preferred_element_type=jnp.float32)
    out_ref[...] = (
        jnp.dot(oh_ref[...], m, preferred_element_type=jnp.float32) + b2_ref[...]
    )


def kernel(prefix, emb, W1, b1, W2, b2):
    B, L = prefix.shape
    V, D = emb.shape
    H = W1.shape[1]
    N = W2.shape[1]
    T = B * L

    TN = 2048
    grid = (N // TN,)

    idx = prefix.reshape(T, 1).astype(jnp.int32)

    out = pl.pallas_call(
        _mlp_body,
        grid=grid,
        in_specs=[
            pl.BlockSpec((T, 1), lambda j: (0, 0)),
            pl.BlockSpec((V, D), lambda j: (0, 0)),
            pl.BlockSpec((D, H), lambda j: (0, 0)),
            pl.BlockSpec((1, H), lambda j: (0, 0)),
            pl.BlockSpec((H, TN), lambda j: (0, j)),
            pl.BlockSpec((1, TN), lambda j: (0, j)),
        ],
        out_specs=pl.BlockSpec((T, TN), lambda j: (0, j)),
        out_shape=jax.ShapeDtypeStruct((T, N), jnp.float32),
        scratch_shapes=[
            pltpu.VMEM((V, H), jnp.float32),
            pltpu.VMEM((T, V), jnp.float32),
        ],
    )(idx, emb, W1, b1.reshape(1, H), W2, b2.reshape(1, N))

    return out.reshape(B, L, N)


# speedup vs baseline: 1.0530x; 1.0219x over previous
import jax
import jax.numpy as jnp
from jax.experimental import pallas as pl
from jax.experimental.pallas import tpu as pltpu


def _mlp_body(idx_ref, emb_ref, w1_ref, b1_ref, w2_ref, b2_ref, out_ref,
              htab_ref, oh_ref):
    T, V = idx_ref.shape[0], emb_ref.shape[0]

    @pl.when(pl.program_id(0) == 0)
    def _compute_h():
        h = jnp.dot(emb_ref[...], w1_ref[...], preferred_element_type=jnp.float32)
        htab_ref[...] = jnp.tanh(h + b1_ref[...])
        iota = jax.lax.broadcasted_iota(jnp.int32, (T, V), 1)
        oh_ref[...] = jnp.where(iota == idx_ref[...], 1.0, 0.0).astype(jnp.float32)

    m = jnp.dot(htab_ref[...], w2_ref[...], preferred_element_type=jnp.float32)
    out_ref[...] = (
        jnp.dot(oh_ref[...], m, preferred_element_type=jnp.float32) + b2_ref[...]
    )


def kernel(prefix, emb, W1, b1, W2, b2):
    B, L = prefix.shape
    V, D = emb.shape
    H = W1.shape[1]
    N = W2.shape[1]
    T = B * L

    TN = 4096
    grid = (N // TN,)

    idx = prefix.reshape(T, 1).astype(jnp.int32)

    out = pl.pallas_call(
        _mlp_body,
        grid=grid,
        in_specs=[
            pl.BlockSpec((T, 1), lambda j: (0, 0)),
            pl.BlockSpec((V, D), lambda j: (0, 0)),
            pl.BlockSpec((D, H), lambda j: (0, 0)),
            pl.BlockSpec((1, H), lambda j: (0, 0)),
            pl.BlockSpec((H, TN), lambda j: (0, j)),
            pl.BlockSpec((1, TN), lambda j: (0, j)),
        ],
        out_specs=pl.BlockSpec((T, TN), lambda j: (0, j)),
        out_shape=jax.ShapeDtypeStruct((T, N), jnp.float32),
        scratch_shapes=[
            pltpu.VMEM((V, H), jnp.float32),
            pltpu.VMEM((T, V), jnp.float32),
        ],
    )(idx, emb, W1, b1.reshape(1, H), W2, b2.reshape(1, N))

    return out.reshape(B, L, N)
